# baseline (speedup 1.0000x reference)
import functools

import jax
import jax.numpy as jnp
from jax.experimental import pallas as pl
from jax.experimental.pallas import tpu as pltpu


def _round_up(x, m):
    return -(-x // m) * m


def _sigmoid(x):
    return 0.5 * (jnp.tanh(0.5 * x) + 1.0)


def _lstm_kernel(vpre_ref, sh_ref, emb_ref, wih_ref, whh_ref, b_ref, wfc_ref,
                 bfc_ref, out_ref, xga_ref, xgb_ref, xt_ref, h_sc, c_sc, *,
                 chunk_steps, bb, n_chunks):
    E = emb_ref.shape[1]
    H = whh_ref.shape[0]
    c_idx = pl.program_id(0)
    rows_chunk = chunk_steps * bb

    @pl.when(c_idx == 0)
    def _():
        h_sc[...] = jnp.zeros_like(h_sc)
        c_sc[...] = jnp.zeros_like(c_sc)

    def gather_project(t, tok0, write_xg):
        tbase = tok0 + t * bb
        row_iota = jax.lax.broadcasted_iota(jnp.int32, (8, E), 0)
        for run in range(bb // 8):
            vals = []
            for k in range(8):
                mi = run * 8 + k
                vp = vpre_ref[tbase + mi]
                c8 = pl.multiple_of(vp << 3, 8)
                chunk = emb_ref[pl.ds(c8, 8), :]
                vals.append(pltpu.roll(chunk, sh_ref[tbase + mi], axis=0))
            for lvl in (1, 2, 4):
                vals = [jnp.where((row_iota & lvl) != 0, hi, lo)
                        for lo, hi in zip(vals[0::2], vals[1::2])]
            xt_ref[run * 8:(run + 1) * 8, :] = vals[0]
        xgb = (jnp.dot(xt_ref[...].astype(jnp.bfloat16), wih_ref[...],
                       preferred_element_type=jnp.float32) + b_ref[...])
        r0 = pl.multiple_of(t * bb, bb)
        write_xg[pl.ds(r0, bb), :] = xgb.astype(jnp.bfloat16)

    def recur_step(t, read_xg):
        r0 = pl.multiple_of(t * bb, bb)
        h = h_sc[...]

        def gate(gi):
            return (read_xg[pl.ds(r0, bb), gi * H:(gi + 1) * H]
                    .astype(jnp.float32)
                    + jnp.dot(h, whh_ref[:, gi * H:(gi + 1) * H],
                              preferred_element_type=jnp.float32))

        i_g = _sigmoid(gate(0))
        g_g = jnp.tanh(gate(2))
        c_new = i_g * g_g
        f_g = _sigmoid(gate(1))
        c_new = c_new + f_g * c_sc[...]
        o_g = _sigmoid(gate(3))
        c_sc[...] = c_new
        h_sc[...] = (o_g * jnp.tanh(c_new)).astype(jnp.bfloat16)

    def run(do_p1, do_p2, read_xg, write_xg):
        tok0 = c_idx * rows_chunk

        def body(t, _):
            if do_p1:
                gather_project(t, tok0, write_xg)
            if do_p2:
                recur_step(t, read_xg)
            return 0

        jax.lax.fori_loop(0, chunk_steps, body, 0)

    @pl.when(c_idx == 0)
    def _():
        run(True, False, None, xga_ref)

    if n_chunks > 1:
        @pl.when((c_idx > 0) & (c_idx < n_chunks) & (c_idx % 2 == 1))
        def _():
            run(True, True, xga_ref, xgb_ref)

        @pl.when((c_idx > 0) & (c_idx < n_chunks) & (c_idx % 2 == 0))
        def _():
            run(True, True, xgb_ref, xga_ref)

    @pl.when(c_idx == n_chunks)
    def _():
        run(False, True, xgb_ref if (n_chunks - 1) % 2 else xga_ref, None)

        logits = (jnp.dot(h_sc[...], wfc_ref[...],
                          preferred_element_type=jnp.float32) + bfc_ref[...])
        out_ref[...] = _sigmoid(logits)


def kernel(token_ids, embedding, w_ih, w_hh, b, w_fc, b_fc):
    B, T = token_ids.shape
    V, E = embedding.shape
    H = w_hh.shape[0]
    O = w_fc.shape[1]

    bb = B
    n_chunks = 8
    while T % n_chunks:
        n_chunks //= 2
    chunk_steps = T // n_chunks

    ids = token_ids.astype(jnp.int32).T
    vpre = (ids >> 3).reshape(-1)
    sh = ((jnp.arange(bb, dtype=jnp.int32)[None, :] - ids) & 7).reshape(-1)

    emb = embedding
    if V % 8:
        emb = jnp.pad(emb, ((0, _round_up(V, 8) - V), (0, 0)))

    wih = w_ih.astype(jnp.bfloat16)
    whh = w_hh.astype(jnp.bfloat16)
    O_pad = max(128, _round_up(O, 128))
    wfc = jnp.pad(w_fc, ((0, 0), (0, O_pad - O))).astype(jnp.bfloat16)
    bfc = jnp.pad(b_fc, ((0, 0), (0, O_pad - O)))

    kfn = functools.partial(_lstm_kernel, chunk_steps=chunk_steps,
                            bb=bb, n_chunks=n_chunks)

    out = pl.pallas_call(
        kfn,
        out_shape=jax.ShapeDtypeStruct((B, O_pad), jnp.float32),
        grid_spec=pltpu.PrefetchScalarGridSpec(
            num_scalar_prefetch=2,
            grid=(n_chunks + 1,),
            in_specs=[
                pl.BlockSpec(emb.shape, lambda c, vp, sh: (0, 0)),
                pl.BlockSpec(wih.shape, lambda c, vp, sh: (0, 0)),
                pl.BlockSpec(whh.shape, lambda c, vp, sh: (0, 0)),
                pl.BlockSpec(b.shape, lambda c, vp, sh: (0, 0)),
                pl.BlockSpec(wfc.shape, lambda c, vp, sh: (0, 0)),
                pl.BlockSpec(bfc.shape, lambda c, vp, sh: (0, 0)),
            ],
            out_specs=pl.BlockSpec((bb, O_pad), lambda c, vp, sh: (0, 0)),
            scratch_shapes=[
                pltpu.VMEM((chunk_steps * bb, 4 * H), jnp.bfloat16),
                pltpu.VMEM((chunk_steps * bb, 4 * H), jnp.bfloat16),
                pltpu.VMEM((bb, E), jnp.float32),
                pltpu.VMEM((bb, H), jnp.bfloat16),
                pltpu.VMEM((bb, H), jnp.float32),
            ],
        ),
        compiler_params=pltpu.CompilerParams(
            dimension_semantics=("arbitrary",),
            vmem_limit_bytes=52 << 20),
    )(vpre, sh, emb, wih, whh, b, wfc, bfc)

    return out[:, :O]


# speedup vs baseline: 1.0160x; 1.0160x over previous
import functools

import jax
import jax.numpy as jnp
from jax.experimental import pallas as pl
from jax.experimental.pallas import tpu as pltpu


def _round_up(x, m):
    return -(-x // m) * m


def _sigmoid(x):
    return 0.5 * (jnp.tanh(0.5 * x) + 1.0)


def _lstm_kernel(vpre_ref, sh_ref, emb_ref, wih_ref, whh_ref, b_ref, wfc_ref,
                 bfc_ref, out_ref, xga_ref, xgb_ref, xt_ref, h_sc, c_sc, *,
                 chunk_steps, bb, n_chunks):
    E = emb_ref.shape[1]
    H = whh_ref.shape[0]
    c_idx = pl.program_id(0)
    rows_chunk = chunk_steps * bb

    @pl.when(c_idx == 0)
    def _():
        h_sc[...] = jnp.zeros_like(h_sc)
        c_sc[...] = jnp.zeros_like(c_sc)

    def gather_project(t, tok0, write_xg):
        tbase = tok0 + t * bb
        row_iota = jax.lax.broadcasted_iota(jnp.int32, (8, E), 0)
        for run in range(bb // 8):
            halves = []
            for hl in range(2):
                acc = None
                for j in range(4):
                    k = hl * 4 + j
                    mi = run * 8 + k
                    vp = vpre_ref[tbase + mi]
                    c8 = pl.multiple_of(vp << 3, 8)
                    chunk = emb_ref[pl.ds(c8, 8), :]
                    rolled = pltpu.roll(chunk, sh_ref[tbase + mi], axis=0)
                    acc = rolled if acc is None else jnp.where(
                        row_iota == k, rolled, acc)
                halves.append(acc)
            xt_ref[run * 8:(run + 1) * 8, :] = jnp.where(
                row_iota < 4, halves[0], halves[1])
        xgb = (jnp.dot(xt_ref[...].astype(jnp.bfloat16), wih_ref[...],
                       preferred_element_type=jnp.float32) + b_ref[...])
        r0 = pl.multiple_of(t * bb, bb)
        write_xg[pl.ds(r0, bb), :] = xgb.astype(jnp.bfloat16)

    def recur_step(t, read_xg):
        r0 = pl.multiple_of(t * bb, bb)
        h = h_sc[...]

        def gate(gi):
            return (read_xg[pl.ds(r0, bb), gi * H:(gi + 1) * H]
                    .astype(jnp.float32)
                    + jnp.dot(h, whh_ref[:, gi * H:(gi + 1) * H],
                              preferred_element_type=jnp.float32))

        i_g = _sigmoid(gate(0))
        g_g = jnp.tanh(gate(2))
        c_new = i_g * g_g
        f_g = _sigmoid(gate(1))
        c_new = c_new + f_g * c_sc[...]
        o_g = _sigmoid(gate(3))
        c_sc[...] = c_new
        h_sc[...] = (o_g * jnp.tanh(c_new)).astype(jnp.bfloat16)

    def run(do_p1, do_p2, read_xg, write_xg):
        tok0 = c_idx * rows_chunk

        def body(t, _):
            if do_p1:
                gather_project(t, tok0, write_xg)
            if do_p2:
                recur_step(t, read_xg)
            return 0

        jax.lax.fori_loop(0, chunk_steps, body, 0)

    @pl.when(c_idx == 0)
    def _():
        run(True, False, None, xga_ref)

    if n_chunks > 1:
        @pl.when((c_idx > 0) & (c_idx < n_chunks) & (c_idx % 2 == 1))
        def _():
            run(True, True, xga_ref, xgb_ref)

        @pl.when((c_idx > 0) & (c_idx < n_chunks) & (c_idx % 2 == 0))
        def _():
            run(True, True, xgb_ref, xga_ref)

    @pl.when(c_idx == n_chunks)
    def _():
        run(False, True, xgb_ref if (n_chunks - 1) % 2 else xga_ref, None)

        logits = (jnp.dot(h_sc[...], wfc_ref[...],
                          preferred_element_type=jnp.float32) + bfc_ref[...])
        out_ref[...] = _sigmoid(logits)


def kernel(token_ids, embedding, w_ih, w_hh, b, w_fc, b_fc):
    B, T = token_ids.shape
    V, E = embedding.shape
    H = w_hh.shape[0]
    O = w_fc.shape[1]

    bb = B
    n_chunks = 20
    while T % n_chunks:
        n_chunks //= 2
    chunk_steps = T // n_chunks

    ids = token_ids.astype(jnp.int32).T
    vpre = (ids >> 3).reshape(-1)
    sh = ((jnp.arange(bb, dtype=jnp.int32)[None, :] - ids) & 7).reshape(-1)

    emb = embedding
    if V % 8:
        emb = jnp.pad(emb, ((0, _round_up(V, 8) - V), (0, 0)))

    wih = w_ih.astype(jnp.bfloat16)
    whh = w_hh.astype(jnp.bfloat16)
    O_pad = max(128, _round_up(O, 128))
    wfc = jnp.pad(w_fc, ((0, 0), (0, O_pad - O))).astype(jnp.bfloat16)
    bfc = jnp.pad(b_fc, ((0, 0), (0, O_pad - O)))

    kfn = functools.partial(_lstm_kernel, chunk_steps=chunk_steps,
                            bb=bb, n_chunks=n_chunks)

    out = pl.pallas_call(
        kfn,
        out_shape=jax.ShapeDtypeStruct((B, O_pad), jnp.float32),
        grid_spec=pltpu.PrefetchScalarGridSpec(
            num_scalar_prefetch=2,
            grid=(n_chunks + 1,),
            in_specs=[
                pl.BlockSpec(emb.shape, lambda c, vp, sh: (0, 0)),
                pl.BlockSpec(wih.shape, lambda c, vp, sh: (0, 0)),
                pl.BlockSpec(whh.shape, lambda c, vp, sh: (0, 0)),
                pl.BlockSpec(b.shape, lambda c, vp, sh: (0, 0)),
                pl.BlockSpec(wfc.shape, lambda c, vp, sh: (0, 0)),
                pl.BlockSpec(bfc.shape, lambda c, vp, sh: (0, 0)),
            ],
            out_specs=pl.BlockSpec((bb, O_pad), lambda c, vp, sh: (0, 0)),
            scratch_shapes=[
                pltpu.VMEM((chunk_steps * bb, 4 * H), jnp.bfloat16),
                pltpu.VMEM((chunk_steps * bb, 4 * H), jnp.bfloat16),
                pltpu.VMEM((bb, E), jnp.float32),
                pltpu.VMEM((bb, H), jnp.bfloat16),
                pltpu.VMEM((bb, H), jnp.float32),
            ],
        ),
        compiler_params=pltpu.CompilerParams(
            dimension_semantics=("arbitrary",),
            vmem_limit_bytes=52 << 20),
    )(vpre, sh, emb, wih, whh, b, wfc, bfc)

    return out[:, :O]
